# baseline (speedup 1.0000x reference)
import functools

import jax
import jax.numpy as jnp
from jax import lax
from jax.experimental import pallas as pl
from jax.experimental.pallas import tpu as pltpu
from jax.experimental.pallas import tpu_sc as plsc


@functools.cache
def _make_gather(L, D):
    info = plsc.get_sparse_core_info()
    NC, NS = info.num_cores, info.num_subcores
    NW = NC * NS
    assert L % NW == 0
    b_per_w = L // NW
    mesh = plsc.VectorSubcoreMesh(core_axis_name="c", subcore_axis_name="s")

    NCH = 4
    C = b_per_w // NCH

    @functools.partial(
        pl.kernel,
        mesh=mesh,
        out_type=jax.ShapeDtypeStruct((1, L, D), jnp.float32),
        scratch_types=[
            pltpu.VMEM((b_per_w,), jnp.int32),
            pltpu.VMEM((b_per_w, D), jnp.float32),
            [pltpu.SemaphoreType.DMA] * NCH,
            pltpu.SemaphoreType.DMA,
        ],
        compiler_params=pltpu.CompilerParams(use_tc_tiling_on_sc=False),
    )
    def gather_kernel(table_hbm, idx_hbm, out_hbm, idx_v, rows_v, gsems, ssem):
        wid = lax.axis_index("s") * NC + lax.axis_index("c")
        base = wid * b_per_w
        pltpu.sync_copy(idx_hbm.at[0, pl.ds(base, b_per_w)], idx_v)
        gathers = []
        for k in range(NCH):
            gathers.append(
                pltpu.async_copy(
                    table_hbm.at[idx_v.at[pl.ds(k * C, C)]],
                    rows_v.at[pl.ds(k * C, C)],
                    gsems[k],
                )
            )
        stores = []
        for k in range(NCH):
            gathers[k].wait()
            stores.append(
                pltpu.async_copy(
                    rows_v.at[pl.ds(k * C, C)],
                    out_hbm.at[0, pl.ds(base + k * C, C)],
                    ssem,
                )
            )
        for k in range(NCH):
            stores[k].wait()

    return gather_kernel


def kernel(x, device, table, pe):
    L = x.shape[1]
    return _make_gather(L, table.shape[1])(table, pe)
